# Initial kernel scaffold; baseline (speedup 1.0000x reference)
import functools

import jax
import jax.numpy as jnp
from jax import lax
from jax.experimental import pallas as pl
from jax.experimental.pallas import tpu as pltpu
from jax.experimental.pallas import tpu_sc as plsc

N = 10000
E = 320000
D = 128
H = D // 2
NC, NS, L = 2, 16, 16

CK = 128
CH = 157
EPT = CH * CK
EPAD = NS * EPT
SINK = N
ACCR = N + 16
HSIZE = 10240
HSTRIDE = HSIZE // NS

_mesh = plsc.VectorSubcoreMesh(
    core_axis_name="c", subcore_axis_name="s", num_cores=NC, num_subcores=NS)


@functools.partial(
    pl.kernel,
    out_type=jax.ShapeDtypeStruct((HSIZE,), jnp.float32),
    mesh=_mesh,
    scratch_types=[
        pltpu.VMEM((CH, CK), jnp.int32),
        pltpu.VMEM((HSIZE,), jnp.float32),
        pltpu.VMEM((HSTRIDE,), jnp.float32),
        pltpu.VMEM((HSTRIDE,), jnp.float32),
        pltpu.VMEM_SHARED((NS, HSIZE), jnp.float32),
        pltpu.SemaphoreType.DMA,
    ],
)
def _deg_kernel(dst_hbm, deg_hbm, dst_v, hist, slab, red, shared, sem):
    c = lax.axis_index("c")
    s = lax.axis_index("s")

    @pl.when(c == 0)
    def _():
        pltpu.sync_copy(dst_hbm.at[s], dst_v)
        zeros = jnp.zeros((L,), jnp.float32)
        ones = jnp.ones((L,), jnp.float32)

        def zero_body(i, _):
            hist[pl.ds(pl.multiple_of(i * L, L), L)] = zeros
            return 0
        lax.fori_loop(0, HSIZE // L, zero_body, 0)

        def chunk_body(j, _):
            for v in range(CK // L):
                idx = dst_v[j, pl.ds(v * L, L)]
                plsc.addupdate_scatter(hist, [idx], ones)
            return 0
        lax.fori_loop(0, CH, chunk_body, 0)

        pltpu.sync_copy(hist, shared.at[s])
        plsc.subcore_barrier()

        base = pl.multiple_of(s * HSTRIDE, HSTRIDE)

        def add_body(i, _):
            o = pl.ds(pl.multiple_of(i * L, L), L)
            red[o] = red[o] + slab[o]
            return 0

        def cp_body(i, _):
            o = pl.ds(pl.multiple_of(i * L, L), L)
            red[o] = slab[o]
            return 0

        for t in range(NS):
            pltpu.sync_copy(shared.at[t, pl.ds(base, HSTRIDE)], slab)
            lax.fori_loop(0, HSTRIDE // L, cp_body if t == 0 else add_body, 0)

        pltpu.sync_copy(red, deg_hbm.at[pl.ds(base, HSTRIDE)])


def _agg_core(t_ref, out_ref, s, src_v, dst_v, gbuf, acc, sem):
    rows = N // NS
    rbase = s * rows
    pltpu.sync_copy(t_ref.at[pl.ds(rbase, rows)], acc.at[pl.ds(rbase, rows)])
    plsc.subcore_barrier()

    def chunk_body(j, _):
        pltpu.async_copy(t_ref.at[src_v.at[j]], gbuf, sem).wait()
        pltpu.sync_copy(gbuf, acc.at[dst_v.at[j]], add=True)
        return 0
    lax.fori_loop(0, CH, chunk_body, 0)

    plsc.subcore_barrier()
    pltpu.sync_copy(acc.at[pl.ds(rbase, rows)], out_ref.at[pl.ds(rbase, rows)])


@functools.partial(
    pl.kernel,
    out_type=(jax.ShapeDtypeStruct((N, H), jnp.float32),
              jax.ShapeDtypeStruct((N, H), jnp.float32)),
    mesh=_mesh,
    scratch_types=[
        pltpu.VMEM((CH, CK), jnp.int32),
        pltpu.VMEM((CH, CK), jnp.int32),
        pltpu.VMEM((CK, H), jnp.float32),
        pltpu.VMEM_SHARED((ACCR, H), jnp.float32),
        pltpu.SemaphoreType.DMA,
    ],
)
def _agg_kernel(ta_hbm, tb_hbm, src_hbm, dst_hbm,
                outa, outb, src_v, dst_v, gbuf, acc, sem):
    c = lax.axis_index("c")
    s = lax.axis_index("s")
    pltpu.sync_copy(src_hbm.at[s], src_v)
    pltpu.sync_copy(dst_hbm.at[s], dst_v)

    @pl.when(c == 0)
    def _():
        _agg_core(ta_hbm, outa, s, src_v, dst_v, gbuf, acc, sem)

    @pl.when(c == 1)
    def _():
        _agg_core(tb_hbm, outb, s, src_v, dst_v, gbuf, acc, sem)


def _mm1_body(x_ref, w_ref, deg_ref, ta_ref, tb_ref):
    dis = lax.rsqrt(deg_ref[...][:N] + 1.0)
    t = jnp.dot(x_ref[...], w_ref[...],
                preferred_element_type=jnp.float32) * dis
    ta_ref[...] = t[:, :H]
    tb_ref[...] = t[:, H:]


def _mid_body(aa_ref, ab_ref, deg_ref, b_ref, w_ref, ta_ref, tb_ref):
    dis = lax.rsqrt(deg_ref[...][:N] + 1.0)
    h = jnp.concatenate([aa_ref[...], ab_ref[...]], axis=1) * dis + b_ref[...]
    h = jnp.maximum(h, 0.0)
    t = jnp.dot(h, w_ref[...], preferred_element_type=jnp.float32) * dis
    ta_ref[...] = t[:, :H]
    tb_ref[...] = t[:, H:]


def _post_body(aa_ref, ab_ref, deg_ref, b_ref, o_ref):
    dis = lax.rsqrt(deg_ref[...][:N] + 1.0)
    o_ref[...] = (jnp.concatenate([aa_ref[...], ab_ref[...]], axis=1) * dis
                  + b_ref[...])


_half_pair = [jax.ShapeDtypeStruct((N, H), jnp.float32),
              jax.ShapeDtypeStruct((N, H), jnp.float32)]
_mm1 = pl.pallas_call(_mm1_body, out_shape=_half_pair)
_mid = pl.pallas_call(_mid_body, out_shape=_half_pair)
_post = pl.pallas_call(
    _post_body, out_shape=jax.ShapeDtypeStruct((N, D), jnp.float32))


def kernel(x, edge_index, W1, b1, W2, b2):
    src = edge_index[0].astype(jnp.int32)
    dst = edge_index[1].astype(jnp.int32)
    pad = EPAD - E
    srcp = jnp.concatenate(
        [src, jnp.zeros((pad,), jnp.int32)]).reshape(NS, CH, CK)
    dstp = jnp.concatenate(
        [dst, jnp.full((pad,), SINK, jnp.int32)]).reshape(NS, CH, CK)

    deg = _deg_kernel(dstp).reshape(HSIZE, 1)
    b1r = b1.reshape(1, D)
    b2r = b2.reshape(1, D)

    t1a, t1b = _mm1(x, W1, deg)
    a1a, a1b = _agg_kernel(t1a, t1b, srcp, dstp)
    t2a, t2b = _mid(a1a, a1b, deg, b1r, W2)
    a2a, a2b = _agg_kernel(t2a, t2b, srcp, dstp)
    return _post(a2a, a2b, deg, b2r)


# speedup vs baseline: 16.2471x; 16.2471x over previous
import functools

import jax
import jax.numpy as jnp
from jax import lax
from jax.experimental import pallas as pl
from jax.experimental.pallas import tpu as pltpu
from jax.experimental.pallas import tpu_sc as plsc

N = 10000
E = 320000
D = 128
H = D // 2
NC, NS, L = 2, 16, 16

CK = 128
CH = 157
EPT = CH * CK
EPAD = NS * EPT
SINK = N
ACCR = N + 16
HSIZE = 10240
HSTRIDE = HSIZE // NS

_mesh = plsc.VectorSubcoreMesh(
    core_axis_name="c", subcore_axis_name="s", num_cores=NC, num_subcores=NS)
_sc_params = pltpu.CompilerParams(
    needs_layout_passes=False, use_tc_tiling_on_sc=False)


@functools.partial(
    pl.kernel,
    out_type=jax.ShapeDtypeStruct((HSIZE,), jnp.float32),
    mesh=_mesh,
    scratch_types=[
        pltpu.VMEM((CH, CK), jnp.int32),
        pltpu.VMEM((HSIZE,), jnp.float32),
        pltpu.VMEM((HSTRIDE,), jnp.float32),
        pltpu.VMEM((HSTRIDE,), jnp.float32),
        pltpu.VMEM_SHARED((NS, HSIZE), jnp.float32),
        pltpu.SemaphoreType.DMA,
    ],
    compiler_params=_sc_params,
)
def _deg_kernel(dst_hbm, deg_hbm, dst_v, hist, slab, red, shared, sem):
    c = lax.axis_index("c")
    s = lax.axis_index("s")

    @pl.when(c == 0)
    def _():
        pltpu.sync_copy(dst_hbm.at[s], dst_v)
        zeros = jnp.zeros((L,), jnp.float32)
        ones = jnp.ones((L,), jnp.float32)

        def zero_body(i, _):
            hist[pl.ds(pl.multiple_of(i * L, L), L)] = zeros
            return 0
        lax.fori_loop(0, HSIZE // L, zero_body, 0)

        def chunk_body(j, _):
            for v in range(CK // L):
                idx = dst_v[j, pl.ds(v * L, L)]
                plsc.addupdate_scatter(hist, [idx], ones)
            return 0
        lax.fori_loop(0, CH, chunk_body, 0)

        pltpu.sync_copy(hist, shared.at[s])
        plsc.subcore_barrier()

        base = pl.multiple_of(s * HSTRIDE, HSTRIDE)

        def add_body(i, _):
            o = pl.ds(pl.multiple_of(i * L, L), L)
            red[o] = red[o] + slab[o]
            return 0

        def cp_body(i, _):
            o = pl.ds(pl.multiple_of(i * L, L), L)
            red[o] = slab[o]
            return 0

        for t in range(NS):
            pltpu.sync_copy(shared.at[t, pl.ds(base, HSTRIDE)], slab)
            lax.fori_loop(0, HSTRIDE // L, cp_body if t == 0 else add_body, 0)

        pltpu.sync_copy(red, deg_hbm.at[pl.ds(base, HSTRIDE)])


def _agg_core(t_ref, out_ref, s, src_v, dst_v, gbuf, acc, sem):
    rows = N // NS
    rbase = s * rows
    pltpu.sync_copy(t_ref.at[pl.ds(rbase, rows)], acc.at[pl.ds(rbase, rows)])
    plsc.subcore_barrier()

    def chunk_body(j, _):
        pltpu.async_copy(t_ref.at[src_v.at[j]], gbuf, sem).wait()
        pltpu.sync_copy(gbuf, acc.at[dst_v.at[j]], add=True)
        return 0
    lax.fori_loop(0, CH, chunk_body, 0)

    plsc.subcore_barrier()
    pltpu.sync_copy(acc.at[pl.ds(rbase, rows)], out_ref.at[pl.ds(rbase, rows)])


@functools.partial(
    pl.kernel,
    out_type=(jax.ShapeDtypeStruct((N, H), jnp.float32),
              jax.ShapeDtypeStruct((N, H), jnp.float32)),
    mesh=_mesh,
    scratch_types=[
        pltpu.VMEM((CH, CK), jnp.int32),
        pltpu.VMEM((CH, CK), jnp.int32),
        pltpu.VMEM((CK, H), jnp.float32),
        pltpu.VMEM_SHARED((ACCR, H), jnp.float32),
        pltpu.SemaphoreType.DMA,
    ],
    compiler_params=_sc_params,
)
def _agg_kernel(ta_hbm, tb_hbm, src_hbm, dst_hbm,
                outa, outb, src_v, dst_v, gbuf, acc, sem):
    c = lax.axis_index("c")
    s = lax.axis_index("s")
    pltpu.sync_copy(src_hbm.at[s], src_v)
    pltpu.sync_copy(dst_hbm.at[s], dst_v)

    @pl.when(c == 0)
    def _():
        _agg_core(ta_hbm, outa, s, src_v, dst_v, gbuf, acc, sem)

    @pl.when(c == 1)
    def _():
        _agg_core(tb_hbm, outb, s, src_v, dst_v, gbuf, acc, sem)


def _mm1_body(x_ref, w_ref, deg_ref, ta_ref, tb_ref):
    dis = lax.rsqrt(deg_ref[...][:N] + 1.0)
    t = jnp.dot(x_ref[...], w_ref[...],
                preferred_element_type=jnp.float32) * dis
    ta_ref[...] = t[:, :H]
    tb_ref[...] = t[:, H:]


def _mid_body(aa_ref, ab_ref, deg_ref, b_ref, w_ref, ta_ref, tb_ref):
    dis = lax.rsqrt(deg_ref[...][:N] + 1.0)
    h = jnp.concatenate([aa_ref[...], ab_ref[...]], axis=1) * dis + b_ref[...]
    h = jnp.maximum(h, 0.0)
    t = jnp.dot(h, w_ref[...], preferred_element_type=jnp.float32) * dis
    ta_ref[...] = t[:, :H]
    tb_ref[...] = t[:, H:]


def _post_body(aa_ref, ab_ref, deg_ref, b_ref, o_ref):
    dis = lax.rsqrt(deg_ref[...][:N] + 1.0)
    o_ref[...] = (jnp.concatenate([aa_ref[...], ab_ref[...]], axis=1) * dis
                  + b_ref[...])


_half_pair = [jax.ShapeDtypeStruct((N, H), jnp.float32),
              jax.ShapeDtypeStruct((N, H), jnp.float32)]
_mm1 = pl.pallas_call(_mm1_body, out_shape=_half_pair)
_mid = pl.pallas_call(_mid_body, out_shape=_half_pair)
_post = pl.pallas_call(
    _post_body, out_shape=jax.ShapeDtypeStruct((N, D), jnp.float32))


def kernel(x, edge_index, W1, b1, W2, b2):
    src = edge_index[0].astype(jnp.int32)
    dst = edge_index[1].astype(jnp.int32)
    pad = EPAD - E
    srcp = jnp.concatenate(
        [src, jnp.zeros((pad,), jnp.int32)]).reshape(NS, CH, CK)
    dstp = jnp.concatenate(
        [dst, jnp.full((pad,), SINK, jnp.int32)]).reshape(NS, CH, CK)

    deg = _deg_kernel(dstp).reshape(HSIZE, 1)
    b1r = b1.reshape(1, D)
    b2r = b2.reshape(1, D)

    t1a, t1b = _mm1(x, W1, deg)
    a1a, a1b = _agg_kernel(t1a, t1b, srcp, dstp)
    t2a, t2b = _mid(a1a, a1b, deg, b1r, W2)
    a2a, a2b = _agg_kernel(t2a, t2b, srcp, dstp)
    return _post(a2a, a2b, deg, b2r)
